# baseline (speedup 1.0000x reference)
import functools

import jax
import jax.numpy as jnp
from jax import lax
from jax.experimental import pallas as pl
from jax.experimental.pallas import tpu as pltpu
from jax.experimental.pallas import tpu_sc as plsc

V = 1_000_000
D = 32
B = 16384
EPS = 1e-8

NC = 2
NS = 16
L = 16

VG = V // 4
BG = B // 4
R = 21504
RG = R // 4
NR = 48
NP = NR // NC
EPB = B // NS
NWIN = EPB // L
CH = 32
NCHMAX = EPB // CH
DUMMY = RG
CPG = RG // NS

_OFS = 64
_CUR = 128


def _body(g_hbm, idx_hbm, sc_hbm, w_hbm, m_hbm, out_w, out_m,
          acc1, acc2, idx_stage, plist,
          slot_d, aux, qof, slot_cur,
          gga, ggb, zeros, s1c, s2c, m_chunk, w_chunk, sc_v, sm,
          sem_w, sem_m, sem_s, sem_t, sem_a, sem_b, sem_x, sem_y):
  c = lax.axis_index("c")
  s = lax.axis_index("s")
  t_lo = s * EPB
  lane = lax.iota(jnp.int32, L)

  pltpu.sync_copy(sc_hbm, sc_v)
  pltpu.sync_copy(idx_hbm.at[pl.ds(t_lo, EPB)], idx_stage)
  lrv = sc_v[pl.ds(0, L)]
  vc_l = sc_v[pl.ds(L, L)][0].astype(jnp.int32) - t_lo

  zv = jnp.zeros((L,), jnp.float32)
  def _z(i, _):
    for h in range(8):
      gga[i, pl.ds(h * L, L)] = zv
      ggb[i, pl.ds(h * L, L)] = zv
      zeros[i, pl.ds(h * L, L)] = zv
    return 0
  lax.fori_loop(0, CH, _z, 0)

  def _zc(i, _):
    sm[i] = jnp.int32(0)
    return 0
  lax.fori_loop(0, NR + 2, _zc, 0)

  def _count(w, _):
    v = idx_stage[pl.ds(w * L, L)]
    for l in range(L):
      r = jnp.minimum(v[l] // R, NR - 1)
      r = jnp.where(w * L + l >= vc_l, NR, r)
      sm[r] = sm[r] + 1
    return 0
  lax.fori_loop(0, NWIN, _count, 0)

  def _prefix(r, a):
    sm[_OFS + r] = a
    sm[_CUR + r] = a
    return a + sm[r]
  lax.fori_loop(0, NR + 1, _prefix, jnp.int32(0))

  def _place(w, _):
    v = idx_stage[pl.ds(w * L, L)]
    for l in range(L):
      val = v[l]
      r0 = jnp.minimum(val // R, NR - 1)
      pos = w * L + l
      r = jnp.where(pos >= vc_l, NR, r0)
      d = sm[_CUR + r]
      sm[_CUR + r] = d + 1
      packed = (val - r0 * R) * 1024 + pos
      dw = lax.shift_right_logical(d, 4) * L
      dl = lax.rem(d, L)
      win = plist[pl.ds(dw, L)]
      plist[pl.ds(dw, L)] = jnp.where(lane == dl, packed, win)
    return 0
  lax.fori_loop(0, NWIN, _place, 0)

  def _pass(p, carry):
    rid = 2 * p + c
    base_g = rid * RG

    cl = jnp.minimum(base_g + s * CPG, VG - CPG)
    cp_w = pltpu.async_copy(w_hbm.at[pl.ds(cl, CPG)], out_w.at[pl.ds(cl, CPG)], sem_w)
    cp_m = pltpu.async_copy(m_hbm.at[pl.ds(cl, CPG)], out_m.at[pl.ds(cl, CPG)], sem_m)

    rid_c = jnp.minimum(rid, NR - 1)
    start = sm[_OFS + rid_c]
    K = (sm[_OFS + rid_c + 1] - start) * jnp.where(rid < NR, 1, 0)
    nch = (K + CH - 1) // CH

    seg0 = plist[pl.ds(start, L)]
    e0 = seg0[0]
    e0s = jnp.clip(lax.shift_right_logical(e0, 12), 0, RG - 1)
    e0q = (e0 & 3) * 8192 + (lax.shift_right_logical(e0, 10) & 3) * 32
    e0p = lax.shift_right_logical(jnp.clip(e0 & 1023, 0, EPB - 1) + t_lo, 2)

    def _c2d(j, _):
      off = j * L
      pv = plist[pl.ds(start + off, L)]
      valid = (lane + off) < K
      sl = lax.shift_right_logical(pv, 12)
      qv = (pv & 3) * 8192 + (lax.shift_right_logical(pv, 10) & 3) * 32
      ps = lax.shift_right_logical((pv & 1023) + t_lo, 2)
      jr = lax.shift_right_logical(j, 1)
      jc = lax.rem(j, 2) * L
      slot_d[jr, pl.ds(jc, L)] = jnp.where(valid, sl, DUMMY)
      aux[jr, pl.ds(jc, L)] = jnp.where(valid, ps, e0p)
      qof[jr, pl.ds(jc, L)] = jnp.where(valid, qv, e0q)
      return 0
    lax.fori_loop(0, nch * (CH // L), _c2d, 0)

    def _pa(r, _):
      for h in range(2):
        slot_cur[pl.ds(h * L, L)] = slot_d[r, pl.ds(h * L, L)]
      pltpu.sync_copy(zeros, acc1.at[slot_cur])
      pltpu.sync_copy(zeros, acc2.at[slot_cur])
      return 0
    lax.fori_loop(0, nch, _pa, 0)

    plsc.subcore_barrier()

    def _pb(r, _):
      pltpu.sync_copy(g_hbm.at[aux.at[r]], m_chunk)
      def _bg(w2, _2):
        qw = qof[r, pl.ds(w2 * L, L)]
        for l in range(L):
          qq = qw[l]
          qg = pl.multiple_of(lax.shift_right_logical(qq, 8), 32)
          qo = pl.multiple_of(qq & 255, 32)
          j = w2 * L + l
          g0 = m_chunk[j, pl.ds(qg, L)]
          g1 = m_chunk[j, pl.ds(qg + L, L)]
          gga[j, pl.ds(qo, L)] = g0
          gga[j, pl.ds(qo + L, L)] = g1
          ggb[j, pl.ds(qo, L)] = g0 * g0
          ggb[j, pl.ds(qo + L, L)] = g1 * g1
        return 0
      lax.fori_loop(0, CH // L, _bg, 0)
      for h in range(2):
        slot_cur[pl.ds(h * L, L)] = slot_d[r, pl.ds(h * L, L)]
      pltpu.sync_copy(gga, acc1.at[slot_cur], add=True)
      pltpu.sync_copy(ggb, acc2.at[slot_cur], add=True)
      def _bz(w2, _2):
        qw = qof[r, pl.ds(w2 * L, L)]
        for l in range(L):
          qo = pl.multiple_of(qw[l] & 255, 32)
          j = w2 * L + l
          gga[j, pl.ds(qo, L)] = zv
          gga[j, pl.ds(qo + L, L)] = zv
          ggb[j, pl.ds(qo, L)] = zv
          ggb[j, pl.ds(qo + L, L)] = zv
        return 0
      lax.fori_loop(0, CH // L, _bz, 0)
      return 0
    lax.fori_loop(0, nch, _pb, 0)

    def _c2d2(j, _):
      off = j * L
      valid = (lane + off) < K
      jr = lax.shift_right_logical(j, 1)
      jc = lax.rem(j, 2) * L
      sd = slot_d[jr, pl.ds(jc, L)]
      slc = jnp.where(valid, sd, e0s)
      slot_d[jr, pl.ds(jc, L)] = slc
      aux[jr, pl.ds(jc, L)] = slc + base_g
      return 0
    lax.fori_loop(0, nch * (CH // L), _c2d2, 0)

    cp_w.wait()
    cp_m.wait()

    plsc.subcore_barrier()

    def _pc(r, _):
      for h in range(2):
        slot_cur[pl.ds(h * L, L)] = slot_d[r, pl.ds(h * L, L)]
      d_s = pltpu.async_copy(acc1.at[slot_cur], s1c, sem_s)
      d_t = pltpu.async_copy(acc2.at[slot_cur], s2c, sem_t)
      d_m = pltpu.async_copy(m_hbm.at[aux.at[r]], m_chunk, sem_a)
      d_w = pltpu.async_copy(w_hbm.at[aux.at[r]], w_chunk, sem_b)
      d_s.wait()
      d_t.wait()
      d_m.wait()
      d_w.wait()
      def _fin(j, _2):
        for q in range(4):
          for h in range(2):
            o = q * 2 * L + h * L
            s1 = s1c[j, pl.ds(o, L)]
            s2 = s2c[j, pl.ds(o, L)]
            mm = m_chunk[j, pl.ds(o, L)] + s2
            ww = w_chunk[j, pl.ds(o, L)]
            x = jnp.maximum(mm, jnp.float32(1e-35))
            yi = (jnp.full((L,), 0x5F3759DF, jnp.int32)
                  - lax.shift_right_logical(lax.bitcast_convert_type(x, jnp.int32), 1))
            y = lax.bitcast_convert_type(yi, jnp.float32)
            half = jnp.float32(0.5) * x
            for _it in range(3):
              y = y * (jnp.float32(1.5) - half * y * y)
            denom = mm * y + jnp.float32(EPS)
            upd = ww - lrv * s1 / denom
            m_chunk[j, pl.ds(o, L)] = mm
            w_chunk[j, pl.ds(o, L)] = upd
        return 0
      lax.fori_loop(0, CH, _fin, 0)
      o_m = pltpu.async_copy(m_chunk, out_m.at[aux.at[r]], sem_x)
      o_w = pltpu.async_copy(w_chunk, out_w.at[aux.at[r]], sem_y)
      o_m.wait()
      o_w.wait()
      return 0
    lax.fori_loop(0, nch, _pc, 0)

    plsc.subcore_barrier()
    return carry

  lax.fori_loop(0, NP, _pass, jnp.int32(0))


_adagrad = functools.partial(
    pl.kernel,
    out_type=(jax.ShapeDtypeStruct((VG, 128), jnp.float32),
              jax.ShapeDtypeStruct((VG, 128), jnp.float32)),
    mesh=plsc.VectorSubcoreMesh(core_axis_name="c", subcore_axis_name="s",
                                num_cores=NC, num_subcores=NS),
    scratch_types=[
        pltpu.VMEM_SHARED((RG + 8, 128), jnp.float32),
        pltpu.VMEM_SHARED((RG + 8, 128), jnp.float32),
        pltpu.VMEM((EPB,), jnp.int32),
        pltpu.VMEM((EPB + CH + L,), jnp.int32),
        pltpu.VMEM((NCHMAX, CH), jnp.int32),
        pltpu.VMEM((NCHMAX, CH), jnp.int32),
        pltpu.VMEM((NCHMAX, CH), jnp.int32),
        pltpu.VMEM((CH,), jnp.int32),
        pltpu.VMEM((CH, 128), jnp.float32),
        pltpu.VMEM((CH, 128), jnp.float32),
        pltpu.VMEM((CH, 128), jnp.float32),
        pltpu.VMEM((CH, 128), jnp.float32),
        pltpu.VMEM((CH, 128), jnp.float32),
        pltpu.VMEM((CH, 128), jnp.float32),
        pltpu.VMEM((CH, 128), jnp.float32),
        pltpu.VMEM((2 * L,), jnp.float32),
        pltpu.SMEM((200,), jnp.int32),
        pltpu.SemaphoreType.DMA,
        pltpu.SemaphoreType.DMA,
        pltpu.SemaphoreType.DMA,
        pltpu.SemaphoreType.DMA,
        pltpu.SemaphoreType.DMA,
        pltpu.SemaphoreType.DMA,
        pltpu.SemaphoreType.DMA,
        pltpu.SemaphoreType.DMA,
    ],
)(_body)


def kernel(gradients, weights, moments, indices, learning_rate, valid_count):
  scalars = jnp.concatenate([
      jnp.full((L,), learning_rate, jnp.float32),
      jnp.full((L,), valid_count, jnp.float32),
  ])
  w2 = jnp.reshape(weights, (VG, 128))
  m2 = jnp.reshape(moments, (VG, 128))
  g2 = jnp.reshape(gradients, (BG, 128))
  ow, om = _adagrad(g2, indices, scalars, w2, m2)
  return jnp.reshape(ow, (V, D)), jnp.reshape(om, (V, D))


# speedup vs baseline: 4.2528x; 4.2528x over previous
import functools

import jax
import jax.numpy as jnp
from jax import lax
from jax.experimental import pallas as pl
from jax.experimental.pallas import tpu as pltpu
from jax.experimental.pallas import tpu_sc as plsc

V = 1_000_000
D = 32
B = 16384
EPS = 1e-8

NC = 2
NS = 16
L = 16

VG = V // 4
BG = B // 4
R = 16384
RG = R // 4
NR = 62
NP = NR // NC
EPB = B // NS
NWIN = EPB // L
CH = 32
NCHMAX = EPB // CH
DUMMY = RG
CPG = RG // NS
CC = 64

_OFS = 64
_CUR = 128


def _body(g_hbm, idx_hbm, sc_hbm, w_hbm, m_hbm, out_w, out_m,
          acc1, acc2, idx_stage, plist,
          slot_d, aux, qof, slot_cur,
          gga, ggb, zeros, s1c, s2c, m_chunk, w_chunk, cbw, cbm, sc_v, sm,
          sem_w, sem_m, sem_s, sem_t, sem_a, sem_b, sem_x, sem_y):
  c = lax.axis_index("c")
  s = lax.axis_index("s")
  t_lo = s * EPB
  lane = lax.iota(jnp.int32, L)

  pltpu.sync_copy(sc_hbm, sc_v)
  pltpu.sync_copy(idx_hbm.at[pl.ds(t_lo, EPB)], idx_stage)
  lrv = sc_v[pl.ds(0, L)]
  vc_l = sc_v[pl.ds(L, L)][0].astype(jnp.int32) - t_lo

  zv = jnp.zeros((L,), jnp.float32)
  def _z(i, _):
    for h in range(8):
      gga[i, pl.ds(h * L, L)] = zv
      ggb[i, pl.ds(h * L, L)] = zv
      zeros[i, pl.ds(h * L, L)] = zv
    return 0
  lax.fori_loop(0, CH, _z, 0)

  def _zc(i, _):
    sm[i] = jnp.int32(0)
    return 0
  lax.fori_loop(0, NR + 2, _zc, 0)

  def _count(w, _):
    v = idx_stage[pl.ds(w * L, L)]
    for l in range(L):
      r = jnp.minimum(v[l] // R, NR - 1)
      r = jnp.where(w * L + l >= vc_l, NR, r)
      sm[r] = sm[r] + 1
    return 0
  lax.fori_loop(0, NWIN, _count, 0)

  def _prefix(r, a):
    sm[_OFS + r] = a
    sm[_CUR + r] = a
    return a + sm[r]
  lax.fori_loop(0, NR + 1, _prefix, jnp.int32(0))

  def _place(w, _):
    v = idx_stage[pl.ds(w * L, L)]
    for l in range(L):
      val = v[l]
      r0 = jnp.minimum(val // R, NR - 1)
      pos = w * L + l
      r = jnp.where(pos >= vc_l, NR, r0)
      d = sm[_CUR + r]
      sm[_CUR + r] = d + 1
      packed = (val - r0 * R) * 1024 + pos
      dw = lax.shift_right_logical(d, 4) * L
      dl = lax.rem(d, L)
      win = plist[pl.ds(dw, L)]
      plist[pl.ds(dw, L)] = jnp.where(lane == dl, packed, win)
    return 0
  lax.fori_loop(0, NWIN, _place, 0)

  def _pass(p, carry):
    rid = 2 * p + c
    base_g = rid * RG

    cl = jnp.minimum(base_g + s * CPG, VG - CPG)
    def _cp(i, _):
      off = cl + i * CC
      gw = pltpu.async_copy(w_hbm.at[pl.ds(off, CC)], cbw, sem_w)
      gm = pltpu.async_copy(m_hbm.at[pl.ds(off, CC)], cbm, sem_m)
      gw.wait()
      gm.wait()
      sw = pltpu.async_copy(cbw, out_w.at[pl.ds(off, CC)], sem_w)
      sm2 = pltpu.async_copy(cbm, out_m.at[pl.ds(off, CC)], sem_m)
      sw.wait()
      sm2.wait()
      return 0
    lax.fori_loop(0, CPG // CC, _cp, 0)

    rid_c = jnp.minimum(rid, NR - 1)
    start = sm[_OFS + rid_c]
    K = (sm[_OFS + rid_c + 1] - start) * jnp.where(rid < NR, 1, 0)
    nch = (K + CH - 1) // CH

    seg0 = plist[pl.ds(start, L)]
    e0 = seg0[0]
    e0s = jnp.clip(lax.shift_right_logical(e0, 12), 0, RG - 1)
    e0q = (e0 & 3) * 8192 + (lax.shift_right_logical(e0, 10) & 3) * 32
    e0p = lax.shift_right_logical(jnp.clip(e0 & 1023, 0, EPB - 1) + t_lo, 2)

    def _c2d(j, _):
      off = j * L
      pv = plist[pl.ds(start + off, L)]
      valid = (lane + off) < K
      sl = lax.shift_right_logical(pv, 12)
      qv = (pv & 3) * 8192 + (lax.shift_right_logical(pv, 10) & 3) * 32
      ps = lax.shift_right_logical((pv & 1023) + t_lo, 2)
      jr = lax.shift_right_logical(j, 1)
      jc = lax.rem(j, 2) * L
      slot_d[jr, pl.ds(jc, L)] = jnp.where(valid, sl, DUMMY)
      aux[jr, pl.ds(jc, L)] = jnp.where(valid, ps, e0p)
      qof[jr, pl.ds(jc, L)] = jnp.where(valid, qv, e0q)
      return 0
    lax.fori_loop(0, nch * (CH // L), _c2d, 0)

    def _pa(r, _):
      for h in range(2):
        slot_cur[pl.ds(h * L, L)] = slot_d[r, pl.ds(h * L, L)]
      pltpu.sync_copy(zeros, acc1.at[slot_cur])
      pltpu.sync_copy(zeros, acc2.at[slot_cur])
      return 0
    lax.fori_loop(0, nch, _pa, 0)

    plsc.subcore_barrier()

    def _pb(r, _):
      pltpu.sync_copy(g_hbm.at[aux.at[r]], m_chunk)
      def _bg(w2, _2):
        qw = qof[r, pl.ds(w2 * L, L)]
        for l in range(L):
          qq = qw[l]
          qg = pl.multiple_of(lax.shift_right_logical(qq, 8), 32)
          qo = pl.multiple_of(qq & 255, 32)
          j = w2 * L + l
          g0 = m_chunk[j, pl.ds(qg, L)]
          g1 = m_chunk[j, pl.ds(qg + L, L)]
          gga[j, pl.ds(qo, L)] = g0
          gga[j, pl.ds(qo + L, L)] = g1
          ggb[j, pl.ds(qo, L)] = g0 * g0
          ggb[j, pl.ds(qo + L, L)] = g1 * g1
        return 0
      lax.fori_loop(0, CH // L, _bg, 0)
      for h in range(2):
        slot_cur[pl.ds(h * L, L)] = slot_d[r, pl.ds(h * L, L)]
      pltpu.sync_copy(gga, acc1.at[slot_cur], add=True)
      pltpu.sync_copy(ggb, acc2.at[slot_cur], add=True)
      def _bz(w2, _2):
        qw = qof[r, pl.ds(w2 * L, L)]
        for l in range(L):
          qo = pl.multiple_of(qw[l] & 255, 32)
          j = w2 * L + l
          gga[j, pl.ds(qo, L)] = zv
          gga[j, pl.ds(qo + L, L)] = zv
          ggb[j, pl.ds(qo, L)] = zv
          ggb[j, pl.ds(qo + L, L)] = zv
        return 0
      lax.fori_loop(0, CH // L, _bz, 0)
      return 0
    lax.fori_loop(0, nch, _pb, 0)

    def _c2d2(j, _):
      off = j * L
      valid = (lane + off) < K
      jr = lax.shift_right_logical(j, 1)
      jc = lax.rem(j, 2) * L
      sd = slot_d[jr, pl.ds(jc, L)]
      slc = jnp.where(valid, sd, e0s)
      slot_d[jr, pl.ds(jc, L)] = slc
      aux[jr, pl.ds(jc, L)] = slc + base_g
      return 0
    lax.fori_loop(0, nch * (CH // L), _c2d2, 0)

    plsc.subcore_barrier()

    def _pc(r, _):
      for h in range(2):
        slot_cur[pl.ds(h * L, L)] = slot_d[r, pl.ds(h * L, L)]
      d_s = pltpu.async_copy(acc1.at[slot_cur], s1c, sem_s)
      d_t = pltpu.async_copy(acc2.at[slot_cur], s2c, sem_t)
      d_m = pltpu.async_copy(m_hbm.at[aux.at[r]], m_chunk, sem_a)
      d_w = pltpu.async_copy(w_hbm.at[aux.at[r]], w_chunk, sem_b)
      d_s.wait()
      d_t.wait()
      d_m.wait()
      d_w.wait()
      def _fin(j, _2):
        for q in range(4):
          for h in range(2):
            o = q * 2 * L + h * L
            s1 = s1c[j, pl.ds(o, L)]
            s2 = s2c[j, pl.ds(o, L)]
            mm = m_chunk[j, pl.ds(o, L)] + s2
            ww = w_chunk[j, pl.ds(o, L)]
            x = jnp.maximum(mm, jnp.float32(1e-35))
            yi = (jnp.full((L,), 0x5F3759DF, jnp.int32)
                  - lax.shift_right_logical(lax.bitcast_convert_type(x, jnp.int32), 1))
            y = lax.bitcast_convert_type(yi, jnp.float32)
            half = jnp.float32(0.5) * x
            for _it in range(3):
              y = y * (jnp.float32(1.5) - half * y * y)
            denom = mm * y + jnp.float32(EPS)
            upd = ww - lrv * s1 / denom
            m_chunk[j, pl.ds(o, L)] = mm
            w_chunk[j, pl.ds(o, L)] = upd
        return 0
      lax.fori_loop(0, CH, _fin, 0)
      o_m = pltpu.async_copy(m_chunk, out_m.at[aux.at[r]], sem_x)
      o_w = pltpu.async_copy(w_chunk, out_w.at[aux.at[r]], sem_y)
      o_m.wait()
      o_w.wait()
      return 0
    lax.fori_loop(0, nch, _pc, 0)

    plsc.subcore_barrier()
    return carry

  lax.fori_loop(0, NP, _pass, jnp.int32(0))


_adagrad = functools.partial(
    pl.kernel,
    out_type=(jax.ShapeDtypeStruct((VG, 128), jnp.float32),
              jax.ShapeDtypeStruct((VG, 128), jnp.float32)),
    mesh=plsc.VectorSubcoreMesh(core_axis_name="c", subcore_axis_name="s",
                                num_cores=NC, num_subcores=NS),
    scratch_types=[
        pltpu.VMEM_SHARED((RG + 8, 128), jnp.float32),
        pltpu.VMEM_SHARED((RG + 8, 128), jnp.float32),
        pltpu.VMEM((EPB,), jnp.int32),
        pltpu.VMEM((EPB + CH + L,), jnp.int32),
        pltpu.VMEM((NCHMAX, CH), jnp.int32),
        pltpu.VMEM((NCHMAX, CH), jnp.int32),
        pltpu.VMEM((NCHMAX, CH), jnp.int32),
        pltpu.VMEM((CH,), jnp.int32),
        pltpu.VMEM((CH, 128), jnp.float32),
        pltpu.VMEM((CH, 128), jnp.float32),
        pltpu.VMEM((CH, 128), jnp.float32),
        pltpu.VMEM((CH, 128), jnp.float32),
        pltpu.VMEM((CH, 128), jnp.float32),
        pltpu.VMEM((CH, 128), jnp.float32),
        pltpu.VMEM((CH, 128), jnp.float32),
        pltpu.VMEM((CC, 128), jnp.float32),
        pltpu.VMEM((CC, 128), jnp.float32),
        pltpu.VMEM((2 * L,), jnp.float32),
        pltpu.SMEM((200,), jnp.int32),
        pltpu.SemaphoreType.DMA,
        pltpu.SemaphoreType.DMA,
        pltpu.SemaphoreType.DMA,
        pltpu.SemaphoreType.DMA,
        pltpu.SemaphoreType.DMA,
        pltpu.SemaphoreType.DMA,
        pltpu.SemaphoreType.DMA,
        pltpu.SemaphoreType.DMA,
    ],
)(_body)


def kernel(gradients, weights, moments, indices, learning_rate, valid_count):
  scalars = jnp.concatenate([
      jnp.full((L,), learning_rate, jnp.float32),
      jnp.full((L,), valid_count, jnp.float32),
  ])
  w2 = jnp.reshape(weights, (VG, 128))
  m2 = jnp.reshape(moments, (VG, 128))
  g2 = jnp.reshape(gradients, (BG, 128))
  ow, om = _adagrad(g2, indices, scalars, w2, m2)
  return jnp.reshape(ow, (V, D)), jnp.reshape(om, (V, D))
